# baseline (speedup 1.0000x reference)
import functools
import math

import jax
import jax.numpy as jnp
from jax import lax
from jax.experimental import pallas as pl
from jax.experimental.pallas import tpu as pltpu
from jax.experimental.pallas import tpu_sc as plsc

_EMBED_DIM = 32
_PAD = 128
_SEQ_LEN = 200
_BATCH = 4096
_B = _BATCH * _SEQ_LEN
_XW = _B // _PAD
_NW = 32
_B_PER_W = _B // _NW
_CHUNK = 3200
_N_CHUNKS = _B_PER_W // _CHUNK
_CROWS = _CHUNK // _PAD
_REPS = _CHUNK // _SEQ_LEN
_SCALE = math.sqrt(_EMBED_DIM)
_H = _EMBED_DIM // 2


@jax.jit
def _tok_embed(x128, table, pe):
    mesh = plsc.VectorSubcoreMesh(core_axis_name="c", subcore_axis_name="s")

    @functools.partial(
        pl.kernel,
        mesh=mesh,
        compiler_params=pltpu.CompilerParams(use_tc_tiling_on_sc=False),
        out_type=jax.ShapeDtypeStruct((_B, _PAD), jnp.float32),
        scratch_types=[
            pltpu.VMEM((_CROWS, _PAD), jnp.int32),
            pltpu.VMEM((_CHUNK, _EMBED_DIM), jnp.float32),
            pltpu.VMEM((_SEQ_LEN, _EMBED_DIM), jnp.float32),
            pltpu.SemaphoreType.DMA,
        ],
    )
    def k(x_hbm, table_hbm, pe_hbm, out_hbm, idx_v, rows_v, pe_v, sem):
        wid = lax.axis_index("s") * 2 + lax.axis_index("c")
        xrow_base = wid * (_B_PER_W // _PAD)
        tok_base = wid * _B_PER_W
        pltpu.sync_copy(pe_hbm.at[0, pl.ds(0, _SEQ_LEN), :], pe_v)

        def chunk_body(g, carry):
            pltpu.sync_copy(x_hbm.at[pl.ds(xrow_base + g * _CROWS, _CROWS), :], idx_v)
            descs = [
                pltpu.async_copy(
                    table_hbm.at[idx_v.at[j]],
                    rows_v.at[pl.ds(j * _PAD, _PAD), :],
                    sem,
                )
                for j in range(_CROWS)
            ]
            for d in descs:
                d.wait()

            def p_body(p, c2):
                pe_lo = pe_v[p, pl.ds(0, _H)]
                pe_hi = pe_v[p, pl.ds(_H, _H)]

                def rep_body(rep, c3):
                    r = rep * _SEQ_LEN + p
                    rows_v[r, pl.ds(0, _H)] = (
                        rows_v[r, pl.ds(0, _H)] * _SCALE + pe_lo
                    )
                    rows_v[r, pl.ds(_H, _H)] = (
                        rows_v[r, pl.ds(_H, _H)] * _SCALE + pe_hi
                    )
                    return c3

                return lax.fori_loop(0, _REPS, rep_body, c2)

            lax.fori_loop(0, _SEQ_LEN, p_body, carry)
            pltpu.sync_copy(
                rows_v,
                out_hbm.at[pl.ds(tok_base + g * _CHUNK, _CHUNK), pl.ds(0, _EMBED_DIM)],
            )
            return carry

        lax.fori_loop(0, _N_CHUNKS, chunk_body, 0)

    return k(x128, table, pe)


def kernel(x, table, pe):
    x128 = x.reshape(_XW, _PAD)
    out_pad = _tok_embed(x128, table, pe)
    out = out_pad.reshape(_BATCH, _SEQ_LEN, _PAD)
    return lax.slice_in_dim(out, 0, _EMBED_DIM, axis=2)
